# baseline (speedup 1.0000x reference)
import jax
import jax.numpy as jnp
from jax import lax
from jax.experimental import pallas as pl
from jax.experimental.pallas import tpu as pltpu
from jax.experimental.pallas import tpu_sc as plsc

S = 1024
D = 128
NC = 2
NS = 16
NW = NC * NS
ROWS = S // NW
JC = 256
NCH = S // JC
CHUNK = JC * D


def _sc_body(x_hbm, rev_hbm, out_hbm, xbuf, obuf):
    wid = lax.axis_index("s") * NC + lax.axis_index("c")
    row0 = wid * ROWS
    for jc in range(NCH):
        pltpu.sync_copy(x_hbm.at[pl.ds(jc * CHUNK, CHUNK)], xbuf)

        def row_body(r, carry, jc=jc):
            i = row0 + r
            start = pl.multiple_of((S - 1 - i + jc * JC) * D, D)
            pltpu.sync_copy(rev_hbm.at[pl.ds(start, CHUNK)], obuf)

            def add_body(k, c2):
                off = pl.multiple_of(k * 128, 128)
                for u in range(8):
                    sl = pl.ds(off + u * 16, 16)
                    plsc.addupdate(obuf.at[sl], xbuf[sl])
                return c2

            lax.fori_loop(0, CHUNK // 128, add_body, 0)
            pltpu.sync_copy(obuf, out_hbm.at[i, pl.ds(jc * CHUNK, CHUNK)])
            return carry

        lax.fori_loop(0, ROWS, row_body, 0)


def kernel(x, table):
    seq = x.shape[1]
    d = x.shape[2]
    assert seq == S and d == D
    rev = jnp.flip(table, axis=0)
    rev = jnp.pad(rev, ((0, 1), (0, 0))).reshape(2 * S * D)
    x_flat = x[0].reshape(S * D)

    mesh = plsc.VectorSubcoreMesh(
        core_axis_name="c", subcore_axis_name="s",
        num_cores=NC, num_subcores=NS,
    )
    sc_fn = pl.kernel(
        _sc_body,
        out_type=jax.ShapeDtypeStruct((S, S * D), jnp.float32),
        mesh=mesh,
        scratch_types=[
            pltpu.VMEM((CHUNK,), jnp.float32),
            pltpu.VMEM((CHUNK,), jnp.float32),
        ],
    )
    out = sc_fn(x_flat, rev)
    return out.reshape(S, S, D)


# speedup vs baseline: 1.7267x; 1.7267x over previous
import jax
import jax.numpy as jnp
from jax import lax
from jax.experimental import pallas as pl
from jax.experimental.pallas import tpu as pltpu
from jax.experimental.pallas import tpu_sc as plsc

S = 1024
D = 128
R_TC = 896
R_SC = S - R_TC
_BI = 8

NC = 2
NS = 16
NW = NC * NS
ROWS = R_SC // NW
JC = 256
NCH = S // JC
CHUNK = JC * D


def _tc_body(x_ref, rev_ref, o_ref):
    i0 = pl.program_id(0) * _BI
    for di in range(_BI):
        start = (S - 1) - (i0 + di)
        o_ref[di] = x_ref[...] + rev_ref[pl.ds(start, S), :]


def _sc_body(x_hbm, rev_hbm, out_hbm, xbuf, obuf):
    wid = lax.axis_index("s") * NC + lax.axis_index("c")
    row0 = wid * ROWS
    for jc in range(NCH):
        pltpu.sync_copy(x_hbm.at[pl.ds(jc * CHUNK, CHUNK)], xbuf)

        def row_body(r, carry, jc=jc):
            i = R_TC + row0 + r
            start = pl.multiple_of((S - 1 - i + jc * JC) * D, D)
            pltpu.sync_copy(rev_hbm.at[pl.ds(start, CHUNK)], obuf)

            def add_body(k, c2):
                off = pl.multiple_of(k * 128, 128)
                for u in range(8):
                    sl = pl.ds(off + u * 16, 16)
                    plsc.addupdate(obuf.at[sl], xbuf[sl])
                return c2

            lax.fori_loop(0, CHUNK // 128, add_body, 0)
            pltpu.sync_copy(obuf, out_hbm.at[row0 + r, pl.ds(jc * CHUNK, CHUNK)])
            return carry

        lax.fori_loop(0, ROWS, row_body, 0)


def kernel(x, table):
    assert x.shape[1] == S and x.shape[2] == D
    rev = jnp.flip(table, axis=0)
    rev = jnp.pad(rev, ((0, 1), (0, 0)))
    x2 = x[0]

    out_tc = pl.pallas_call(
        _tc_body,
        grid=(R_TC // _BI,),
        in_specs=[
            pl.BlockSpec((S, D), lambda i: (0, 0)),
            pl.BlockSpec((2 * S, D), lambda i: (0, 0)),
        ],
        out_specs=pl.BlockSpec((_BI, S, D), lambda i: (i, 0, 0)),
        out_shape=jax.ShapeDtypeStruct((R_TC, S, D), x.dtype),
    )(x2, rev)

    mesh = plsc.VectorSubcoreMesh(
        core_axis_name="c", subcore_axis_name="s",
        num_cores=NC, num_subcores=NS,
    )
    sc_fn = pl.kernel(
        _sc_body,
        out_type=jax.ShapeDtypeStruct((R_SC, S * D), jnp.float32),
        mesh=mesh,
        scratch_types=[
            pltpu.VMEM((CHUNK,), jnp.float32),
            pltpu.VMEM((CHUNK,), jnp.float32),
        ],
    )
    out_sc = sc_fn(x2.reshape(S * D), rev.reshape(2 * S * D))

    return jnp.concatenate([out_tc, out_sc.reshape(R_SC, S, D)], axis=0)


# speedup vs baseline: 6.8779x; 3.9832x over previous
import jax
import jax.numpy as jnp
from jax.experimental import pallas as pl
from jax.experimental.pallas import tpu as pltpu

_BI = 16


def _row_block_kernel(x_ref, rev_ref, o_ref):
    i0 = pl.program_id(0) * _BI
    seq = x_ref.shape[0]
    for di in range(_BI):
        start = (seq - 1) - (i0 + di)
        o_ref[di] = x_ref[...] + rev_ref[pl.ds(start, seq), :]


def kernel(x, table):
    seq = x.shape[1]
    d = x.shape[2]
    maxlen = (table.shape[0] + 1) // 2
    assert maxlen == seq
    rev = jnp.flip(table, axis=0)
    rev = jnp.pad(rev, ((0, 1), (0, 0)))
    x2 = x[0]

    out = pl.pallas_call(
        _row_block_kernel,
        grid=(seq // _BI,),
        in_specs=[
            pl.BlockSpec((seq, d), lambda i: (0, 0)),
            pl.BlockSpec((2 * seq, d), lambda i: (0, 0)),
        ],
        out_specs=pl.BlockSpec((_BI, seq, d), lambda i: (i, 0, 0)),
        out_shape=jax.ShapeDtypeStruct((seq, seq, d), x.dtype),
    )(x2, rev)
    return out
